# baseline (device time: 93173 ns/iter reference)
import jax
import jax.numpy as jnp
from jax import lax
from jax.experimental import pallas as pl
from jax.experimental.pallas import tpu as pltpu

N_DEV = 8
N_STAGES = 3
N_LAYERS = 3
N_EXCH = N_STAGES + N_LAYERS * N_STAGES


def kernel(x, Win0, Wout0, Win1, Wout1, Win2, Wout2):
    b_per, d = x.shape
    b_full = N_DEV * b_per

    def body(x_ref, win0_ref, wout0_ref, win1_ref, wout1_ref, win2_ref,
             wout2_ref, out_ref, xfull, accbuf, recv, send_sems, recv_sems):
        me = lax.axis_index("i")

        xfull[pl.ds(me * b_per, b_per), :] = x_ref[:, :]
        for s in range(N_STAGES):
            sz = (1 << s) * b_per
            base = ((me >> s) << s) * b_per
            partner = me ^ (1 << s)
            rdma = pltpu.make_async_remote_copy(
                src_ref=xfull.at[pl.ds(base, sz)],
                dst_ref=xfull.at[pl.ds(base, sz)],
                send_sem=send_sems.at[s],
                recv_sem=recv_sems.at[s],
                device_id=(partner,),
                device_id_type=pl.DeviceIdType.MESH,
            )
            rdma.start()
            rdma.wait()

        xcur = xfull[:, :]
        for k, (win, wout) in enumerate(
            [(win0_ref, wout0_ref), (win1_ref, wout1_ref), (win2_ref, wout2_ref)]
        ):
            h = jnp.maximum(
                jnp.dot(xcur, win[:, :], preferred_element_type=jnp.float32), 0.0
            )
            accbuf[:, :] = jnp.dot(
                h, wout[:, :], preferred_element_type=jnp.float32
            )
            for s in range(N_STAGES):
                idx = N_STAGES + k * N_STAGES + s
                partner = me ^ (1 << s)
                rdma = pltpu.make_async_remote_copy(
                    src_ref=accbuf,
                    dst_ref=recv.at[k * N_STAGES + s],
                    send_sem=send_sems.at[idx],
                    recv_sem=recv_sems.at[idx],
                    device_id=(partner,),
                    device_id_type=pl.DeviceIdType.MESH,
                )
                rdma.start()
                rdma.wait()
                accbuf[:, :] = accbuf[:, :] + recv[k * N_STAGES + s]
            xcur = accbuf[:, :]

        out_ref[:, :] = xcur

    return pl.pallas_call(
        body,
        out_shape=jax.ShapeDtypeStruct((b_full, d), jnp.float32),
        in_specs=[pl.BlockSpec(memory_space=pltpu.VMEM)] * 7,
        out_specs=pl.BlockSpec(memory_space=pltpu.VMEM),
        scratch_shapes=[
            pltpu.VMEM((b_full, d), jnp.float32),
            pltpu.VMEM((b_full, d), jnp.float32),
            pltpu.VMEM((N_LAYERS * N_STAGES, b_full, d), jnp.float32),
            pltpu.SemaphoreType.DMA((N_EXCH,)),
            pltpu.SemaphoreType.DMA((N_EXCH,)),
        ],
    )(x, Win0, Wout0, Win1, Wout1, Win2, Wout2)


# device time: 86407 ns/iter; 1.0783x vs baseline; 1.0783x over previous
import jax
import jax.numpy as jnp
from jax import lax
from jax.experimental import pallas as pl
from jax.experimental.pallas import tpu as pltpu

N_DEV = 8
N_LAYERS = 3
RS_STAGES = [(4, 2, 4), (3, 1, 2), (1, 0, 1)]
AG_STAGES = [(1, 1), (3, 2), (4, 4)]
N_EXCH = 3 + N_LAYERS * 6


def kernel(x, Win0, Wout0, Win1, Wout1, Win2, Wout2):
    bp, d = x.shape
    b_full = N_DEV * bp

    def body(x_ref, win0_ref, wout0_ref, win1_ref, wout1_ref, win2_ref,
             wout2_ref, out_ref, xfull, accbuf, recv_rs, send_sems, recv_sems):
        me = lax.axis_index("i")

        def ag_stage(buf, s, idx):
            mask, blocks = AG_STAGES[s]
            sz = blocks * bp
            base = ((me >> s) << s) * bp
            rdma = pltpu.make_async_remote_copy(
                src_ref=buf.at[pl.ds(base, sz)],
                dst_ref=buf.at[pl.ds(base, sz)],
                send_sem=send_sems.at[idx],
                recv_sem=recv_sems.at[idx],
                device_id=(me ^ mask,),
                device_id_type=pl.DeviceIdType.MESH,
            )
            rdma.start()
            rdma.wait()

        xfull[pl.ds(me * bp, bp), :] = x_ref[:, :]
        for s in range(3):
            ag_stage(xfull, s, s)

        xcur = xfull[:, :]
        for k, (win, wout) in enumerate(
            [(win0_ref, wout0_ref), (win1_ref, wout1_ref), (win2_ref, wout2_ref)]
        ):
            h = jnp.maximum(
                jnp.dot(xcur, win[:, :], preferred_element_type=jnp.float32), 0.0
            )
            accbuf[:, :] = jnp.dot(
                h, wout[:, :], preferred_element_type=jnp.float32
            )

            for s, (mask, bit, half_b) in enumerate(RS_STAGES):
                idx = 3 + k * 6 + s
                half = half_b * bp
                prev_base = ((me >> (bit + 1)) << (bit + 1)) * bp
                bitval = (me >> bit) & 1
                keep_base = prev_base + bitval * half
                send_base = prev_base + (1 - bitval) * half
                rdma = pltpu.make_async_remote_copy(
                    src_ref=accbuf.at[pl.ds(send_base, half)],
                    dst_ref=recv_rs.at[k * 3 + s, pl.ds(0, half)],
                    send_sem=send_sems.at[idx],
                    recv_sem=recv_sems.at[idx],
                    device_id=(me ^ mask,),
                    device_id_type=pl.DeviceIdType.MESH,
                )
                rdma.start()
                rdma.wait()
                accbuf[pl.ds(keep_base, half), :] = (
                    accbuf[pl.ds(keep_base, half), :]
                    + recv_rs[k * 3 + s, pl.ds(0, half), :]
                )

            for s in range(3):
                ag_stage(accbuf, s, 3 + k * 6 + 3 + s)
            xcur = accbuf[:, :]

        out_ref[:, :] = xcur

    return pl.pallas_call(
        body,
        out_shape=jax.ShapeDtypeStruct((b_full, d), jnp.float32),
        in_specs=[pl.BlockSpec(memory_space=pltpu.VMEM)] * 7,
        out_specs=pl.BlockSpec(memory_space=pltpu.VMEM),
        scratch_shapes=[
            pltpu.VMEM((b_full, d), jnp.float32),
            pltpu.VMEM((b_full, d), jnp.float32),
            pltpu.VMEM((N_LAYERS * 3, 4 * bp, d), jnp.float32),
            pltpu.SemaphoreType.DMA((N_EXCH,)),
            pltpu.SemaphoreType.DMA((N_EXCH,)),
        ],
    )(x, Win0, Wout0, Win1, Wout1, Win2, Wout2)


# device time: 56979 ns/iter; 1.6352x vs baseline; 1.5165x over previous
import jax
import jax.numpy as jnp
from jax import lax
from jax.experimental import pallas as pl
from jax.experimental.pallas import tpu as pltpu

N_DEV = 8
N_LAYERS = 3
GROUPS = [(0, 384), (384, 384), (768, 256)]
SCHEDULES = [(1, 3, 4), (3, 4, 1), (4, 1, 3)]


def kernel(x, Win0, Wout0, Win1, Wout1, Win2, Wout2):
    bp, d = x.shape
    b_full = N_DEV * bp

    def body(x_ref, win0_ref, wout0_ref, win1_ref, wout1_ref, win2_ref,
             wout2_ref, out_ref, xfull, accbuf, recv_fe,
             bsend_sems, brecv_sems, send_sems, recv_sems):
        me = lax.axis_index("i")

        xfull[pl.ds(me * bp, bp), :] = x_ref[:, :]
        rdmas = []
        for j in range(1, N_DEV):
            peer = me ^ j
            rdma = pltpu.make_async_remote_copy(
                src_ref=xfull.at[pl.ds(me * bp, bp)],
                dst_ref=xfull.at[pl.ds(me * bp, bp)],
                send_sem=bsend_sems.at[j],
                recv_sem=brecv_sems.at[j],
                device_id=(peer,),
                device_id_type=pl.DeviceIdType.MESH,
            )
            rdma.start()
            rdmas.append(rdma)
        for j in range(1, N_DEV):
            peer = me ^ j
            recv = pltpu.make_async_remote_copy(
                src_ref=xfull.at[pl.ds(peer * bp, bp)],
                dst_ref=xfull.at[pl.ds(peer * bp, bp)],
                send_sem=bsend_sems.at[j],
                recv_sem=brecv_sems.at[j],
                device_id=(peer,),
                device_id_type=pl.DeviceIdType.MESH,
            )
            recv.wait_recv()
        for rdma in rdmas:
            rdma.wait_send()

        xcur = xfull[:, :]
        for k, (win, wout) in enumerate(
            [(win0_ref, wout0_ref), (win1_ref, wout1_ref), (win2_ref, wout2_ref)]
        ):
            h = jnp.maximum(
                jnp.dot(xcur, win[:, :], preferred_element_type=jnp.float32), 0.0
            )
            accbuf[:, :] = jnp.dot(
                h, wout[:, :], preferred_element_type=jnp.float32
            )

            for s in range(3):
                buf = k * 3 + s
                stage_rdmas = []
                for g, (roff, rows) in enumerate(GROUPS):
                    idx = buf * 3 + g
                    mask = SCHEDULES[g][s]
                    rdma = pltpu.make_async_remote_copy(
                        src_ref=accbuf.at[pl.ds(roff, rows)],
                        dst_ref=recv_fe.at[buf, pl.ds(roff, rows)],
                        send_sem=send_sems.at[idx],
                        recv_sem=recv_sems.at[idx],
                        device_id=(me ^ mask,),
                        device_id_type=pl.DeviceIdType.MESH,
                    )
                    rdma.start()
                    stage_rdmas.append(rdma)
                for rdma in stage_rdmas:
                    rdma.wait()
                accbuf[:, :] = accbuf[:, :] + recv_fe[buf, :, :]
            xcur = accbuf[:, :]

        out_ref[:, :] = xcur

    return pl.pallas_call(
        body,
        out_shape=jax.ShapeDtypeStruct((b_full, d), jnp.float32),
        in_specs=[pl.BlockSpec(memory_space=pltpu.VMEM)] * 7,
        out_specs=pl.BlockSpec(memory_space=pltpu.VMEM),
        scratch_shapes=[
            pltpu.VMEM((b_full, d), jnp.float32),
            pltpu.VMEM((b_full, d), jnp.float32),
            pltpu.VMEM((N_LAYERS * 3, b_full, d), jnp.float32),
            pltpu.SemaphoreType.DMA((N_DEV,)),
            pltpu.SemaphoreType.DMA((N_DEV,)),
            pltpu.SemaphoreType.DMA((N_LAYERS * 9,)),
            pltpu.SemaphoreType.DMA((N_LAYERS * 9,)),
        ],
    )(x, Win0, Wout0, Win1, Wout1, Win2, Wout2)


# device time: 39426 ns/iter; 2.3632x vs baseline; 1.4452x over previous
import jax
import jax.numpy as jnp
from jax import lax
from jax.experimental import pallas as pl
from jax.experimental.pallas import tpu as pltpu

N_DEV = 8
N_LAYERS = 3
SCHEDULES = [(1, 3, 4), (3, 4, 1), (4, 1, 3)]
CHUNKS = [
    (0, 176, 0), (176, 176, 0),
    (352, 176, 1), (528, 176, 1),
    (704, 160, 2), (864, 160, 2),
]
CHUNK_BLOCKS = [
    [0, 1], [1, 2], [2, 3, 4], [4, 5], [5, 6], [6, 7],
]
N_FE_SEMS = N_LAYERS * 3 * len(CHUNKS)


def kernel(x, Win0, Wout0, Win1, Wout1, Win2, Wout2):
    bp, d = x.shape
    b_full = N_DEV * bp

    def body(x_ref, win0_ref, wout0_ref, win1_ref, wout1_ref, win2_ref,
             wout2_ref, out_ref, xfull, accs, sendb, recv_fe,
             bsend_sems, brecv_sems, send_sems, recv_sems):
        me = lax.axis_index("i")
        wins = [win0_ref, win1_ref, win2_ref]
        wouts = [wout0_ref, wout1_ref, wout2_ref]

        barrier_sem = pltpu.get_barrier_semaphore()
        for j in range(1, N_DEV):
            pl.semaphore_signal(
                barrier_sem, inc=1,
                device_id=(me ^ j,), device_id_type=pl.DeviceIdType.MESH,
            )
        pl.semaphore_wait(barrier_sem, N_DEV - 1)

        xfull[pl.ds(me * bp, bp), :] = x_ref[:, :].astype(jnp.bfloat16)
        bcast_rdmas = []
        for j in range(1, N_DEV):
            peer = me ^ j
            rdma = pltpu.make_async_remote_copy(
                src_ref=xfull.at[pl.ds(me * bp, bp)],
                dst_ref=xfull.at[pl.ds(me * bp, bp)],
                send_sem=bsend_sems.at[j],
                recv_sem=brecv_sems.at[j],
                device_id=(peer,),
                device_id_type=pl.DeviceIdType.MESH,
            )
            rdma.start()
            bcast_rdmas.append(rdma)

        def wait_block(b):
            @pl.when(me != b)
            def _():
                recv = pltpu.make_async_remote_copy(
                    src_ref=xfull.at[pl.ds(b * bp, bp)],
                    dst_ref=xfull.at[pl.ds(b * bp, bp)],
                    send_sem=bsend_sems.at[0],
                    recv_sem=brecv_sems.at[me ^ b],
                    device_id=(b,),
                    device_id_type=pl.DeviceIdType.MESH,
                )
                recv.wait_recv()

        def fe_rdma(k, s, ci):
            roff, rows, g = CHUNKS[ci]
            idx = (k * 3 + s) * len(CHUNKS) + ci
            return pltpu.make_async_remote_copy(
                src_ref=sendb.at[k * 3 + s, pl.ds(roff, rows)],
                dst_ref=recv_fe.at[k * 3 + s, pl.ds(roff, rows)],
                send_sem=send_sems.at[idx],
                recv_sem=recv_sems.at[idx],
                device_id=(me ^ SCHEDULES[g][s],),
                device_id_type=pl.DeviceIdType.MESH,
            )

        def compute(k, ci):
            roff, rows, _ = CHUNKS[ci]
            if k == 0:
                xin = xfull[pl.ds(roff, rows), :].astype(jnp.float32)
            else:
                xin = (
                    accs[k - 1, pl.ds(roff, rows), :]
                    + recv_fe[(k - 1) * 3 + 2, pl.ds(roff, rows), :].astype(
                        jnp.float32
                    )
                )
            h = jnp.maximum(
                jnp.dot(xin, wins[k][:, :], preferred_element_type=jnp.float32),
                0.0,
            )
            p = jnp.dot(h, wouts[k][:, :], preferred_element_type=jnp.float32)
            accs[k, pl.ds(roff, rows), :] = p
            sendb[k * 3, pl.ds(roff, rows), :] = p.astype(jnp.bfloat16)

        pending = {}
        seen_blocks = set()
        for ci in range(len(CHUNKS)):
            for b in CHUNK_BLOCKS[ci]:
                if b not in seen_blocks:
                    seen_blocks.add(b)
                    wait_block(b)
            compute(0, ci)
            pending[ci] = fe_rdma(0, 0, ci)
            pending[ci].start()
        for k in range(N_LAYERS):
            for s in range(3):
                for ci, (roff, rows, g) in enumerate(CHUNKS):
                    pending[ci].wait()
                    if s < 2:
                        a = (
                            accs[k, pl.ds(roff, rows), :]
                            + recv_fe[k * 3 + s, pl.ds(roff, rows), :].astype(
                                jnp.float32
                            )
                        )
                        accs[k, pl.ds(roff, rows), :] = a
                        sendb[k * 3 + s + 1, pl.ds(roff, rows), :] = a.astype(
                            jnp.bfloat16
                        )
                        pending[ci] = fe_rdma(k, s + 1, ci)
                        pending[ci].start()
                    elif k < N_LAYERS - 1:
                        compute(k + 1, ci)
                        pending[ci] = fe_rdma(k + 1, 0, ci)
                        pending[ci].start()
                    else:
                        out_ref[pl.ds(roff, rows), :] = (
                            accs[k, pl.ds(roff, rows), :]
                            + recv_fe[k * 3 + 2, pl.ds(roff, rows), :].astype(
                                jnp.float32
                            )
                        )
        for rdma in bcast_rdmas:
            rdma.wait_send()

    return pl.pallas_call(
        body,
        out_shape=jax.ShapeDtypeStruct((b_full, d), jnp.float32),
        in_specs=[pl.BlockSpec(memory_space=pltpu.VMEM)] * 7,
        out_specs=pl.BlockSpec(memory_space=pltpu.VMEM),
        scratch_shapes=[
            pltpu.VMEM((b_full, d), jnp.bfloat16),
            pltpu.VMEM((N_LAYERS, b_full, d), jnp.float32),
            pltpu.VMEM((N_LAYERS * 3, b_full, d), jnp.bfloat16),
            pltpu.VMEM((N_LAYERS * 3, b_full, d), jnp.bfloat16),
            pltpu.SemaphoreType.DMA((N_DEV,)),
            pltpu.SemaphoreType.DMA((N_DEV,)),
            pltpu.SemaphoreType.DMA((N_FE_SEMS,)),
            pltpu.SemaphoreType.DMA((N_FE_SEMS,)),
        ],
        compiler_params=pltpu.CompilerParams(collective_id=0),
    )(x, Win0, Wout0, Win1, Wout1, Win2, Wout2)
